# baseline (speedup 1.0000x reference)
import functools

import jax
import jax.numpy as jnp
from jax import lax
from jax.experimental import pallas as pl
from jax.experimental.pallas import tpu as pltpu
from jax.experimental.pallas import tpu_sc as plsc

EMBED = 64
PADDED = 128
NC = 2
NS = 16
NW = NC * NS

NBUF = 8


@functools.cache
def _make(BATCH, SEQ):
    assert BATCH % NW == 0
    r_per_w = BATCH // NW
    n_idx = r_per_w * SEQ
    CH = 40
    nperrow = SEQ // CH
    nch = nperrow * r_per_w
    assert nch % NBUF == 0
    nsteps = nch // NBUF
    mesh = plsc.VectorSubcoreMesh(core_axis_name="c", subcore_axis_name="s")

    @functools.partial(
        pl.kernel,
        mesh=mesh,
        out_type=jax.ShapeDtypeStruct((BATCH, SEQ, PADDED), jnp.float32),
        scratch_types=[
            pltpu.VMEM((n_idx,), jnp.int32),
            *[pltpu.VMEM((CH, PADDED), jnp.float32) for _ in range(NBUF)],
            *[pltpu.SemaphoreType.DMA for _ in range(2 * NBUF)],
        ],
        compiler_params=pltpu.CompilerParams(use_tc_tiling_on_sc=True),
    )
    def k(x_hbm, table_hbm, out_hbm, idx_v, *bufs):
        rows = bufs[:NBUF]
        gsem = bufs[NBUF:2 * NBUF]
        psem = bufs[2 * NBUF:]
        wid = lax.axis_index("s") * NC + lax.axis_index("c")
        base = wid * r_per_w
        pltpu.sync_copy(x_hbm.at[pl.ds(base * SEQ, n_idx)], idx_v)

        def gather(g, b):
            pltpu.async_copy(
                table_hbm.at[idx_v.at[pl.ds(g * CH, CH)]], rows[b], gsem[b]
            )

        def wait_gather(g, b):
            pltpu.make_async_copy(
                table_hbm.at[idx_v.at[pl.ds(g * CH, CH)]], rows[b], gsem[b]
            ).wait()

        def _out_slice(g, b):
            r = g // nperrow
            c = g - r * nperrow
            return out_hbm.at[base + r, pl.ds(c * CH, CH), :]

        def put(g, b):
            pltpu.async_copy(rows[b], _out_slice(g, b), psem[b])

        def wait_put(g, b):
            pltpu.make_async_copy(rows[b], _out_slice(g, b), psem[b]).wait()

        for b in range(NBUF):
            gather(b, b)

        def body(s, carry):
            for b in range(NBUF):
                g = s * NBUF + b
                wait_gather(g, b)
                put(g, b)
                wait_put(g, b)
                gather(g + NBUF, b)
            return carry

        lax.fori_loop(0, nsteps - 1, body, 0)

        for b in range(NBUF):
            g = (nsteps - 1) * NBUF + b
            wait_gather(g, b)
            put(g, b)
            wait_put(g, b)

    return k


TBLK = 32768


@functools.cache
def _make_transpose_pad(V):
    def body(tt_ref, out_ref):
        out_ref[:, :EMBED] = jnp.transpose(tt_ref[...])

    return pl.pallas_call(
        body,
        grid=((V + TBLK - 1) // TBLK,),
        in_specs=[pl.BlockSpec((EMBED, TBLK), lambda i: (0, i))],
        out_specs=pl.BlockSpec((TBLK, PADDED), lambda i: (i, 0)),
        out_shape=jax.ShapeDtypeStruct((V, PADDED), jnp.float32),
        compiler_params=pltpu.CompilerParams(
            dimension_semantics=("arbitrary",)
        ),
    )


def kernel(x, table):
    bsz, seq = x.shape
    x_flat = x.reshape(bsz * seq).astype(jnp.int32)
    table_p = _make_transpose_pad(table.shape[0])(table.T)
    out_p = _make(bsz, seq)(x_flat, table_p)
    return out_p[:, :, :EMBED]
